# Initial kernel scaffold; baseline (speedup 1.0000x reference)
import jax
import jax.numpy as jnp
from jax.experimental import pallas as pl


def kernel(x_user, x_item, edge_index_buys, edge_index_bought, W_buys, att_src_buys, att_dst_buys, bias_buys, W_bought, att_src_bought, att_dst_bought, bias_bought):
    raise NotImplementedError("write your pallas kernel here")


# speedup vs baseline: 14.8796x; 14.8796x over previous
import functools

import jax
import jax.numpy as jnp
from jax import lax
from jax.experimental import pallas as pl
from jax.experimental.pallas import tpu as pltpu
from jax.experimental.pallas import tpu_sc as plsc

H = 4
C = 32
D = H * C
NC, NS, LANES = 2, 16, 16
NW = NC * NS
EB = 128
TBW = C + 16
ABW = 16
RB = 1000


def _src_tbl_body(x_ref, w2_ref, out_ref):
    out_ref[0] = jnp.dot(x_ref[...], w2_ref[0],
                         preferred_element_type=jnp.float32)


def _a_tbl_body(x_ref, u_ref, out_ref):
    out_ref[...] = jnp.dot(x_ref[...], u_ref[...],
                           preferred_element_type=jnp.float32)


def _post_body(h0, h1, h2, h3, den_ref, bias_ref, out_ref):
    den = den_ref[0] + den_ref[1]
    cols = []
    for h, href in enumerate((h0, h1, h2, h3)):
        s = href[0] + href[1]
        d = den[:, h:h + 1] + 1e-16
        cols.append(s / d)
    out = jnp.concatenate(cols, axis=1) + bias_ref[...]
    out_ref[...] = jnp.where(out > 0, out, jnp.exp(out) - 1.0)


def _src_tbl(x, w2):
    n = x.shape[0]
    return pl.pallas_call(
        _src_tbl_body,
        grid=(n // RB, H),
        in_specs=[pl.BlockSpec((RB, D), lambda i, h: (i, 0)),
                  pl.BlockSpec((1, D, TBW), lambda i, h: (h, 0, 0))],
        out_specs=pl.BlockSpec((1, RB, TBW), lambda i, h: (h, i, 0)),
        out_shape=jax.ShapeDtypeStruct((H, n, TBW), jnp.float32),
    )(x, w2)


def _a_tbl(x, u):
    n = x.shape[0]
    return pl.pallas_call(
        _a_tbl_body,
        grid=(n // RB,),
        in_specs=[pl.BlockSpec((RB, D), lambda i: (i, 0)),
                  pl.BlockSpec((D, ABW), lambda i: (0, 0))],
        out_specs=pl.BlockSpec((RB, ABW), lambda i: (i, 0)),
        out_shape=jax.ShapeDtypeStruct((n, ABW), jnp.float32),
    )(x, u)


def _post(parts, den, bias, n):
    return pl.pallas_call(
        _post_body,
        grid=(n // RB,),
        in_specs=[pl.BlockSpec((NC, RB, C), lambda i: (0, i, 0))] * H
                 + [pl.BlockSpec((NC, RB, ABW), lambda i: (0, i, 0)),
                    pl.BlockSpec((1, D), lambda i: (0, 0))],
        out_specs=pl.BlockSpec((RB, D), lambda i: (i, 0)),
        out_shape=jax.ShapeDtypeStruct((n, D), jnp.float32),
    )(*parts, den, bias.reshape(1, D))


def _make_sc_denom(npad, e, ew):
    nch = ew // EB
    nps = npad // NS
    mesh = plsc.VectorSubcoreMesh(core_axis_name="c", subcore_axis_name="s")

    def body(asrc_tbl, adst_tbl, sidx_hbm, didx_hbm, zeros_hbm, out_hbm,
             sidx_v, didx_v, asr_v, adr_v, srow_v, acc, sem1, sem2):
        cid = lax.axis_index("c")
        sid = lax.axis_index("s")
        wid = sid * NC + cid
        pltpu.sync_copy(zeros_hbm.at[pl.ds(sid * nps, nps)],
                        acc.at[pl.ds(sid * nps, nps)])
        plsc.subcore_barrier()

        def chunk(ci, _):
            base = wid * ew + ci * EB
            pltpu.sync_copy(sidx_hbm.at[pl.ds(base, EB)], sidx_v)
            pltpu.sync_copy(didx_hbm.at[pl.ds(base, EB)], didx_v)
            cp1 = pltpu.async_copy(asrc_tbl.at[sidx_v], asr_v, sem1)
            cp2 = pltpu.async_copy(adst_tbl.at[didx_v], adr_v, sem2)
            cp1.wait()
            cp2.wait()

            def edge(ei, _):
                al = asr_v[ei] + adr_v[ei]
                al = jnp.where(al > 0, al, 0.2 * al)
                s = jnp.exp(al)
                s = s * (base + ei < e).astype(jnp.float32)
                srow_v[ei] = s
                return 0

            lax.fori_loop(0, EB, edge, 0)
            pltpu.sync_copy(srow_v, acc.at[didx_v], add=True)
            return 0

        lax.fori_loop(0, nch, chunk, 0)
        plsc.subcore_barrier()
        pltpu.sync_copy(acc.at[pl.ds(sid * nps, nps)],
                        out_hbm.at[cid, pl.ds(sid * nps, nps)])

    return pl.kernel(
        body,
        out_type=jax.ShapeDtypeStruct((NC, npad, ABW), jnp.float32),
        mesh=mesh,
        compiler_params=pltpu.CompilerParams(use_tc_tiling_on_sc=False),
        scratch_types=[
            pltpu.VMEM((EB,), jnp.int32),
            pltpu.VMEM((EB,), jnp.int32),
            pltpu.VMEM((EB, ABW), jnp.float32),
            pltpu.VMEM((EB, ABW), jnp.float32),
            pltpu.VMEM((EB, ABW), jnp.float32),
            pltpu.VMEM_SHARED((npad, ABW), jnp.float32),
            pltpu.SemaphoreType.DMA,
            pltpu.SemaphoreType.DMA,
        ],
    )


def _make_sc_head(npad, e, ew, h):
    nch = ew // EB
    nps = npad // NS
    mesh = plsc.VectorSubcoreMesh(core_axis_name="c", subcore_axis_name="s")

    def body(stbl, adst_tbl, sidx_hbm, didx_hbm, zeros_hbm, out_hbm,
             sidx_v, didx_v, rows_v, adr_v, msg_v, acc, sem1, sem2):
        cid = lax.axis_index("c")
        sid = lax.axis_index("s")
        wid = sid * NC + cid
        pltpu.sync_copy(zeros_hbm.at[pl.ds(sid * nps, nps)],
                        acc.at[pl.ds(sid * nps, nps)])
        plsc.subcore_barrier()

        def chunk(ci, _):
            base = wid * ew + ci * EB
            pltpu.sync_copy(sidx_hbm.at[pl.ds(base, EB)], sidx_v)
            pltpu.sync_copy(didx_hbm.at[pl.ds(base, EB)], didx_v)
            cp1 = pltpu.async_copy(stbl.at[sidx_v], rows_v, sem1)
            cp2 = pltpu.async_copy(adst_tbl.at[didx_v], adr_v, sem2)
            cp1.wait()
            cp2.wait()

            def edge(ei, _):
                al = rows_v[ei, pl.ds(C, LANES)] + adr_v[ei]
                al = jnp.where(al > 0, al, 0.2 * al)
                s = jnp.exp(al)
                sh = s[h] * (base + ei < e).astype(jnp.float32)
                msg_v[ei, pl.ds(0, LANES)] = (
                    rows_v[ei, pl.ds(0, LANES)] * sh)
                msg_v[ei, pl.ds(LANES, LANES)] = (
                    rows_v[ei, pl.ds(LANES, LANES)] * sh)
                return 0

            lax.fori_loop(0, EB, edge, 0)
            pltpu.sync_copy(msg_v, acc.at[didx_v], add=True)
            return 0

        lax.fori_loop(0, nch, chunk, 0)
        plsc.subcore_barrier()
        pltpu.sync_copy(acc.at[pl.ds(sid * nps, nps)],
                        out_hbm.at[cid, pl.ds(sid * nps, nps)])

    return pl.kernel(
        body,
        out_type=jax.ShapeDtypeStruct((NC, npad, C), jnp.float32),
        mesh=mesh,
        compiler_params=pltpu.CompilerParams(use_tc_tiling_on_sc=False),
        scratch_types=[
            pltpu.VMEM((EB,), jnp.int32),
            pltpu.VMEM((EB,), jnp.int32),
            pltpu.VMEM((EB, TBW), jnp.float32),
            pltpu.VMEM((EB, ABW), jnp.float32),
            pltpu.VMEM((EB, C), jnp.float32),
            pltpu.VMEM_SHARED((npad, C), jnp.float32),
            pltpu.SemaphoreType.DMA,
            pltpu.SemaphoreType.DMA,
        ],
    )


def _make_u(w, att):
    wh = w.reshape(D, H, C)
    u = jnp.einsum("dhc,hc->dh", wh, att)
    return jnp.pad(u, ((0, 0), (0, ABW - H)))


def _gat_type(x_src, x_dst, ei, w, att_src, att_dst, bias):
    n_dst = x_dst.shape[0]
    e = ei.shape[1]
    ew = -(-e // (NW * EB)) * EB
    ep = ew * NW

    u_s = _make_u(w, att_src)
    u_d = _make_u(w, att_dst)
    w2 = jnp.stack([
        jnp.concatenate(
            [w[:, h * C:(h + 1) * C], u_s], axis=1)
        for h in range(H)
    ])

    tbl = _src_tbl(x_src, w2)
    a_s = _a_tbl(x_src, u_s)
    a_d = _a_tbl(x_dst, u_d)

    pad = ep - e
    src_p = jnp.concatenate([ei[0], jnp.zeros((pad,), ei.dtype)])
    dst_p = jnp.concatenate([ei[1], jnp.zeros((pad,), ei.dtype)])
    npad = -(-n_dst // (NS * 8)) * (NS * 8)
    zeros_a = jnp.zeros((npad, ABW), jnp.float32)
    zeros_c = jnp.zeros((npad, C), jnp.float32)

    den = _make_sc_denom(npad, e, ew)(a_s, a_d, src_p, dst_p, zeros_a)
    parts = [
        _make_sc_head(npad, e, ew, h)(tbl[h], a_d, src_p, dst_p, zeros_c)
        for h in range(H)
    ]
    return _post(parts, den, bias, n_dst)


def kernel(x_user, x_item, edge_index_buys, edge_index_bought,
           W_buys, att_src_buys, att_dst_buys, bias_buys,
           W_bought, att_src_bought, att_dst_bought, bias_bought):
    out_item = _gat_type(x_user, x_item, edge_index_buys,
                         W_buys, att_src_buys, att_dst_buys, bias_buys)
    out_user = _gat_type(x_item, x_user, edge_index_bought,
                         W_bought, att_src_bought, att_dst_bought,
                         bias_bought)
    return (out_user, out_item)


# speedup vs baseline: 16.4349x; 1.1045x over previous
import functools

import jax
import jax.numpy as jnp
from jax import lax
from jax.experimental import pallas as pl
from jax.experimental.pallas import tpu as pltpu
from jax.experimental.pallas import tpu_sc as plsc

H = 4
C = 32
D = H * C
NC, NS, LANES = 2, 16, 16
NW = NC * NS
EB = 128
SCH = 8
TBW = C + 16
ABW = 16
RB = 1000


def _src_tbl_body(x_ref, w2_ref, out_ref):
    out_ref[0] = jnp.dot(x_ref[...], w2_ref[0],
                         preferred_element_type=jnp.float32)


def _a_tbl_body(x_ref, u_ref, out_ref):
    out_ref[...] = jnp.dot(x_ref[...], u_ref[...],
                           preferred_element_type=jnp.float32)


def _post_body(h0, h1, h2, h3, den_ref, bias_ref, out_ref):
    den = den_ref[0] + den_ref[1]
    cols = []
    for h, href in enumerate((h0, h1, h2, h3)):
        s = href[0] + href[1]
        d = den[:, h:h + 1] + 1e-16
        cols.append(s / d)
    out = jnp.concatenate(cols, axis=1) + bias_ref[...]
    out_ref[...] = jnp.where(out > 0, out, jnp.exp(out) - 1.0)


def _src_tbl(x, w2):
    n = x.shape[0]
    return pl.pallas_call(
        _src_tbl_body,
        grid=(n // RB, H),
        in_specs=[pl.BlockSpec((RB, D), lambda i, h: (i, 0)),
                  pl.BlockSpec((1, D, TBW), lambda i, h: (h, 0, 0))],
        out_specs=pl.BlockSpec((1, RB, TBW), lambda i, h: (h, i, 0)),
        out_shape=jax.ShapeDtypeStruct((H, n, TBW), jnp.float32),
    )(x, w2)


def _a_tbl(x, u):
    n = x.shape[0]
    return pl.pallas_call(
        _a_tbl_body,
        grid=(n // RB,),
        in_specs=[pl.BlockSpec((RB, D), lambda i: (i, 0)),
                  pl.BlockSpec((D, ABW), lambda i: (0, 0))],
        out_specs=pl.BlockSpec((RB, ABW), lambda i: (i, 0)),
        out_shape=jax.ShapeDtypeStruct((n, ABW), jnp.float32),
    )(x, u)


def _post(parts, den, bias, n):
    return pl.pallas_call(
        _post_body,
        grid=(n // RB,),
        in_specs=[pl.BlockSpec((NC, RB, C), lambda i: (0, i, 0))] * H
                 + [pl.BlockSpec((NC, RB, ABW), lambda i: (0, i, 0)),
                    pl.BlockSpec((1, D), lambda i: (0, 0))],
        out_specs=pl.BlockSpec((RB, D), lambda i: (i, 0)),
        out_shape=jax.ShapeDtypeStruct((n, D), jnp.float32),
    )(*parts, den, bias.reshape(1, D))


def _make_sc_denom(npad, e, ew):
    nch = ew // EB
    nps = npad // NS
    mesh = plsc.VectorSubcoreMesh(core_axis_name="c", subcore_axis_name="s")

    def body(asrc_tbl, adst_tbl, sidx_hbm, didx_hbm, zeros_hbm, out_hbm,
             sidx_v, didx_v, asr_v, adr_v, srow_v, acc, sem1, sem2):
        cid = lax.axis_index("c")
        sid = lax.axis_index("s")
        wid = sid * NC + cid
        pltpu.sync_copy(zeros_hbm.at[pl.ds(sid * nps, nps)],
                        acc.at[pl.ds(sid * nps, nps)])
        plsc.subcore_barrier()

        def chunk(ci, _):
            base = wid * ew + ci * EB
            pltpu.sync_copy(sidx_hbm.at[pl.ds(base, EB)], sidx_v)
            pltpu.sync_copy(didx_hbm.at[pl.ds(base, EB)], didx_v)
            cp1 = pltpu.async_copy(asrc_tbl.at[sidx_v], asr_v, sem1)
            cp2 = pltpu.async_copy(adst_tbl.at[didx_v], adr_v, sem2)
            cp1.wait()
            cp2.wait()

            def edge(ei, _):
                al = asr_v[ei] + adr_v[ei]
                al = jnp.where(al > 0, al, 0.2 * al)
                srow_v[ei] = jnp.exp(al)
                return 0

            lax.fori_loop(0, EB, edge, 0, unroll=4)
            pltpu.sync_copy(srow_v, acc.at[didx_v], add=True)
            return 0

        lax.fori_loop(0, nch, chunk, 0)
        plsc.subcore_barrier()
        pltpu.sync_copy(acc.at[pl.ds(sid * nps, nps)],
                        out_hbm.at[cid, pl.ds(sid * nps, nps)])

    return pl.kernel(
        body,
        out_type=jax.ShapeDtypeStruct((NC, npad, ABW), jnp.float32),
        mesh=mesh,
        compiler_params=pltpu.CompilerParams(use_tc_tiling_on_sc=False),
        scratch_types=[
            pltpu.VMEM((EB,), jnp.int32),
            pltpu.VMEM((EB,), jnp.int32),
            pltpu.VMEM((EB, ABW), jnp.float32),
            pltpu.VMEM((EB, ABW), jnp.float32),
            pltpu.VMEM((EB, ABW), jnp.float32),
            pltpu.VMEM_SHARED((npad, ABW), jnp.float32),
            pltpu.SemaphoreType.DMA,
            pltpu.SemaphoreType.DMA,
        ],
    )


def _make_sc_head(npad, e, ew, h):
    nch = ew // EB
    nps = npad // NS
    mesh = plsc.VectorSubcoreMesh(core_axis_name="c", subcore_axis_name="s")

    def body(stbl, adst_tbl, sidx_hbm, didx_hbm, zeros_hbm, out_hbm,
             sidx_v, didx_v, rows_v, adr_v, msg_v, acc, sem1, sem2):
        cid = lax.axis_index("c")
        sid = lax.axis_index("s")
        wid = sid * NC + cid
        pltpu.sync_copy(zeros_hbm.at[pl.ds(sid * nps, nps)],
                        acc.at[pl.ds(sid * nps, nps)])
        plsc.subcore_barrier()

        def chunk(ci, _):
            base = wid * ew + ci * EB
            pltpu.sync_copy(sidx_hbm.at[pl.ds(base, EB)], sidx_v)
            pltpu.sync_copy(didx_hbm.at[pl.ds(base, EB)], didx_v)
            cp1 = pltpu.async_copy(stbl.at[sidx_v], rows_v, sem1)
            cp2 = pltpu.async_copy(adst_tbl.at[didx_v], adr_v, sem2)
            cp1.wait()
            cp2.wait()

            def edge(ei, _):
                al = rows_v[ei, pl.ds(C, LANES)] + adr_v[ei]
                al = jnp.where(al > 0, al, 0.2 * al)
                s = jnp.exp(al)
                sh = s[h]
                msg_v[ei, pl.ds(0, LANES)] = (
                    rows_v[ei, pl.ds(0, LANES)] * sh)
                msg_v[ei, pl.ds(LANES, LANES)] = (
                    rows_v[ei, pl.ds(LANES, LANES)] * sh)
                return 0

            lax.fori_loop(0, EB, edge, 0, unroll=4)
            pltpu.sync_copy(msg_v, acc.at[didx_v], add=True)
            return 0

        lax.fori_loop(0, nch, chunk, 0)
        plsc.subcore_barrier()
        pltpu.sync_copy(acc.at[pl.ds(sid * nps, nps)],
                        out_hbm.at[cid, pl.ds(sid * nps, nps)])

    return pl.kernel(
        body,
        out_type=jax.ShapeDtypeStruct((NC, npad, C), jnp.float32),
        mesh=mesh,
        compiler_params=pltpu.CompilerParams(use_tc_tiling_on_sc=False),
        scratch_types=[
            pltpu.VMEM((EB,), jnp.int32),
            pltpu.VMEM((EB,), jnp.int32),
            pltpu.VMEM((EB, TBW), jnp.float32),
            pltpu.VMEM((EB, ABW), jnp.float32),
            pltpu.VMEM((EB, C), jnp.float32),
            pltpu.VMEM_SHARED((npad, C), jnp.float32),
            pltpu.SemaphoreType.DMA,
            pltpu.SemaphoreType.DMA,
        ],
    )


def _make_u(w, att):
    wh = w.reshape(D, H, C)
    u = jnp.einsum("dhc,hc->dh", wh, att)
    return jnp.pad(u, ((0, 0), (0, ABW - H)))


def _gat_type(x_src, x_dst, ei, w, att_src, att_dst, bias):
    n_dst = x_dst.shape[0]
    e = ei.shape[1]
    ew = -(-e // (NW * EB)) * EB
    ep = ew * NW

    u_s = _make_u(w, att_src)
    u_d = _make_u(w, att_dst)
    w2 = jnp.stack([
        jnp.concatenate(
            [w[:, h * C:(h + 1) * C], u_s], axis=1)
        for h in range(H)
    ])

    tbl = _src_tbl(x_src, w2)
    a_s = _a_tbl(x_src, u_s)
    a_d = _a_tbl(x_dst, u_d)

    pad = ep - e
    npad = -(-(n_dst + 1) // (NS * 8)) * (NS * 8)
    src_p = jnp.concatenate([ei[0], jnp.zeros((pad,), ei.dtype)])
    dst_p = jnp.concatenate([ei[1], jnp.full((pad,), n_dst, ei.dtype)])
    zeros_a = jnp.zeros((npad, ABW), jnp.float32)
    zeros_c = jnp.zeros((npad, C), jnp.float32)

    den = _make_sc_denom(npad, e, ew)(a_s, a_d, src_p, dst_p, zeros_a)
    parts = [
        _make_sc_head(npad, e, ew, h)(tbl[h], a_d, src_p, dst_p, zeros_c)
        for h in range(H)
    ]
    return _post(parts, den, bias, n_dst)


def kernel(x_user, x_item, edge_index_buys, edge_index_bought,
           W_buys, att_src_buys, att_dst_buys, bias_buys,
           W_bought, att_src_bought, att_dst_bought, bias_bought):
    out_item = _gat_type(x_user, x_item, edge_index_buys,
                         W_buys, att_src_buys, att_dst_buys, bias_buys)
    out_user = _gat_type(x_item, x_user, edge_index_bought,
                         W_bought, att_src_bought, att_dst_bought,
                         bias_bought)
    return (out_user, out_item)
